# Initial kernel scaffold; baseline (speedup 1.0000x reference)
import jax
import jax.numpy as jnp
from jax.experimental import pallas as pl


def kernel(x, edge_index, batch, W1l, b1, W1r, W2l, b2, W2r):
    raise NotImplementedError("write your pallas kernel here")


# speedup vs baseline: 11.2059x; 11.2059x over previous
import jax
import jax.numpy as jnp
from jax import lax
from jax.experimental import pallas as pl
from jax.experimental.pallas import tpu as pltpu
from jax.experimental.pallas import tpu_sc as plsc

NC = 2
NS = 16
NW = NC * NS
K = 80
CW = 16


_SC_PARAMS = pltpu.CompilerParams(use_tc_tiling_on_sc=False)


def _make_agg(n, d, nchunk_w):
    mesh = plsc.VectorSubcoreMesh(core_axis_name="c", subcore_axis_name="s")
    out_type = [jax.ShapeDtypeStruct((NC, n, d), jnp.float32)]
    scratch = [
        pltpu.VMEM((nchunk_w, K), jnp.int32),
        pltpu.VMEM((nchunk_w, K), jnp.int32),
        pltpu.VMEM((2, K, d), jnp.float32),
        pltpu.VMEM_SHARED((n, d), jnp.float32),
        pltpu.SemaphoreType.DMA,
        pltpu.SemaphoreType.DMA,
    ]

    def body(y_hbm, src_hbm, dst_hbm, zacc_hbm, acc_out,
             sidx, didx, rows, acc_sh, sem0, sem1):
        c = lax.axis_index("c")
        s = lax.axis_index("s")
        wid = c * NS + s

        @pl.when(s == 0)
        def _():
            pltpu.sync_copy(zacc_hbm, acc_sh)

        pltpu.sync_copy(src_hbm.at[wid], sidx)
        pltpu.sync_copy(dst_hbm.at[wid], didx)
        plsc.subcore_barrier()

        sems = (sem0, sem1)

        def start(j, b):
            pltpu.async_copy(y_hbm.at[sidx.at[j]], rows.at[b], sems[b])

        def wait(j, b):
            pltpu.make_async_copy(y_hbm.at[sidx.at[j]], rows.at[b],
                                  sems[b]).wait()

        def drain(j, b):
            pltpu.sync_copy(rows.at[b], acc_sh.at[didx.at[j]], add=True)

        start(0, 0)

        def step(it, carry):
            for b in range(2):
                j = it * 2 + b
                start(j + 1, 1 - b)
                wait(j, b)
                drain(j, b)
            return carry

        lax.fori_loop(0, (nchunk_w - 1) // 2, step, 0)
        jlast = nchunk_w - 1
        wait(jlast, jlast % 2)
        drain(jlast, jlast % 2)

        plsc.subcore_barrier()
        rpt = (n // NS) // 8 * 8
        tail = n - NS * rpt
        pltpu.sync_copy(acc_sh.at[pl.ds(s * rpt, rpt)],
                        acc_out.at[c].at[pl.ds(s * rpt, rpt)])
        if tail:
            @pl.when(s == NS - 1)
            def _():
                pltpu.sync_copy(acc_sh.at[pl.ds(NS * rpt, tail)],
                                acc_out.at[c].at[pl.ds(NS * rpt, tail)])

    return pl.kernel(body, out_type=out_type, mesh=mesh,
                     scratch_types=scratch, compiler_params=_SC_PARAMS)


def _make_cnt(n, nchunk_w):
    mesh = plsc.VectorSubcoreMesh(core_axis_name="c", subcore_axis_name="s")
    out_type = [jax.ShapeDtypeStruct((NC, n, CW), jnp.float32)]
    scratch = [
        pltpu.VMEM((nchunk_w, K), jnp.int32),
        pltpu.VMEM((K, CW), jnp.float32),
        pltpu.VMEM_SHARED((n, CW), jnp.float32),
    ]

    def body(dst_hbm, zcnt_hbm, ones_hbm, cnt_out, didx, ones_v, cnt_sh):
        c = lax.axis_index("c")
        s = lax.axis_index("s")
        wid = c * NS + s

        @pl.when(s == 0)
        def _():
            pltpu.sync_copy(zcnt_hbm, cnt_sh)

        pltpu.sync_copy(ones_hbm, ones_v)
        pltpu.sync_copy(dst_hbm.at[wid], didx)
        plsc.subcore_barrier()

        def step(j, carry):
            pltpu.sync_copy(ones_v, cnt_sh.at[didx.at[j]], add=True)
            return carry

        lax.fori_loop(0, nchunk_w, step, 0)

        plsc.subcore_barrier()
        rpt = (n // NS) // 8 * 8
        tail = n - NS * rpt
        pltpu.sync_copy(cnt_sh.at[pl.ds(s * rpt, rpt)],
                        cnt_out.at[c].at[pl.ds(s * rpt, rpt)])
        if tail:
            @pl.when(s == NS - 1)
            def _():
                pltpu.sync_copy(cnt_sh.at[pl.ds(NS * rpt, tail)],
                                cnt_out.at[c].at[pl.ds(NS * rpt, tail)])

    return pl.kernel(body, out_type=out_type, mesh=mesh,
                     scratch_types=scratch, compiler_params=_SC_PARAMS)


def _tc_pre(x, wl_t, wr_t, b):
    n, d = x.shape
    blk = 1000
    grid = n // blk

    def body(x_ref, wl_ref, wr_ref, b_ref, y_ref, r_ref):
        xb = x_ref[...]
        y_ref[...] = jnp.dot(xb, wl_ref[...],
                             preferred_element_type=jnp.float32)
        r_ref[...] = jnp.dot(xb, wr_ref[...],
                             preferred_element_type=jnp.float32) + b_ref[...]

    return pl.pallas_call(
        body,
        grid=(grid,),
        in_specs=[pl.BlockSpec((blk, d), lambda i: (i, 0)),
                  pl.BlockSpec((d, d), lambda i: (0, 0)),
                  pl.BlockSpec((d, d), lambda i: (0, 0)),
                  pl.BlockSpec((1, d), lambda i: (0, 0))],
        out_specs=[pl.BlockSpec((blk, d), lambda i: (i, 0)),
                   pl.BlockSpec((blk, d), lambda i: (i, 0))],
        out_shape=[jax.ShapeDtypeStruct((n, d), jnp.float32)] * 2,
    )(x, wl_t, wr_t, b.reshape(1, d))


def _tc_mid(s1, cnt, r1, wl_t, wr_t, b):
    _, n, d = s1.shape
    blk = 1000
    grid = n // blk

    def body(s_ref, c_ref, r_ref, wl_ref, wr_ref, b_ref, y_ref, r2_ref):
        ssum = s_ref[0] + s_ref[1]
        csum = c_ref[0] + c_ref[1]
        cnt_col = csum[:, 0:1]
        mean = ssum / jnp.clip(cnt_col, 1.0, None)
        h = jnp.maximum(mean + r_ref[...], 0.0)
        y_ref[...] = jnp.dot(h, wl_ref[...],
                             preferred_element_type=jnp.float32)
        r2_ref[...] = jnp.dot(h, wr_ref[...],
                              preferred_element_type=jnp.float32) + b_ref[...]

    return pl.pallas_call(
        body,
        grid=(grid,),
        in_specs=[pl.BlockSpec((NC, blk, d), lambda i: (0, i, 0)),
                  pl.BlockSpec((NC, blk, CW), lambda i: (0, i, 0)),
                  pl.BlockSpec((blk, d), lambda i: (i, 0)),
                  pl.BlockSpec((d, d), lambda i: (0, 0)),
                  pl.BlockSpec((d, d), lambda i: (0, 0)),
                  pl.BlockSpec((1, d), lambda i: (0, 0))],
        out_specs=[pl.BlockSpec((blk, d), lambda i: (i, 0)),
                   pl.BlockSpec((blk, d), lambda i: (i, 0))],
        out_shape=[jax.ShapeDtypeStruct((n, d), jnp.float32)] * 2,
    )(s1, cnt, r1, wl_t, wr_t, b.reshape(1, d))


def _tc_post(s2, cnt, r2, batch3, g):
    _, n, d = s2.shape
    blk = 1000
    grid = n // blk

    def body(s_ref, c_ref, r_ref, b_ref, out_ref, acc_ref, csum_ref):
        i = pl.program_id(0)
        ssum = s_ref[0] + s_ref[1]
        csum = c_ref[0] + c_ref[1]
        cnt_col = csum[:, 0:1]
        out_nodes = ssum / jnp.clip(cnt_col, 1.0, None) + r_ref[...]
        bids = b_ref[0]
        gid = lax.broadcasted_iota(jnp.int32, (g, blk), 0)
        mask = (gid == bids).astype(jnp.float32)

        @pl.when(i == 0)
        def _():
            acc_ref[...] = jnp.zeros_like(acc_ref)
            csum_ref[...] = jnp.zeros_like(csum_ref)

        acc_ref[...] += jnp.dot(mask, out_nodes,
                                preferred_element_type=jnp.float32)
        csum_ref[...] += jnp.broadcast_to(
            jnp.sum(mask, axis=1, keepdims=True), (g, d))

        @pl.when(i == grid - 1)
        def _():
            out_ref[...] = acc_ref[...] / jnp.clip(csum_ref[...], 1.0, None)

    return pl.pallas_call(
        body,
        grid=(grid,),
        in_specs=[pl.BlockSpec((NC, blk, d), lambda i: (0, i, 0)),
                  pl.BlockSpec((NC, blk, CW), lambda i: (0, i, 0)),
                  pl.BlockSpec((blk, d), lambda i: (i, 0)),
                  pl.BlockSpec((1, 1, blk), lambda i: (i, 0, 0))],
        out_specs=pl.BlockSpec((g, d), lambda i: (0, 0)),
        out_shape=jax.ShapeDtypeStruct((g, d), jnp.float32),
        scratch_shapes=[pltpu.VMEM((g, d), jnp.float32),
                        pltpu.VMEM((g, d), jnp.float32)],
    )(s2, cnt, r2, batch3)


def kernel(x, edge_index, batch, W1l, b1, W1r, W2l, b2, W2r):
    x = x.astype(jnp.float32)
    n, d = x.shape
    e = edge_index.shape[1]
    g = 64
    assert e % (NW * K) == 0
    nrows = e // K
    nchunk_w = nrows // NW

    src = edge_index[0].astype(jnp.int32).reshape(NW, nchunk_w, K)
    dst = edge_index[1].astype(jnp.int32).reshape(NW, nchunk_w, K)
    batch3 = batch.astype(jnp.int32).reshape(n // 1000, 1, 1000)

    zacc = jnp.zeros((n, d), jnp.float32)
    zcnt = jnp.zeros((n, CW), jnp.float32)
    ones = jnp.ones((K, CW), jnp.float32)

    agg = _make_agg(n, d, nchunk_w)
    cnt_kernel = _make_cnt(n, nchunk_w)

    y1, r1 = _tc_pre(x, W1l.T, W1r.T, b1)
    (cnt,) = cnt_kernel(dst, zcnt, ones)
    (s1,) = agg(y1, src, dst, zacc)
    y2, r2 = _tc_mid(s1, cnt, r1, W2l.T, W2r.T, b2)
    (s2,) = agg(y2, src, dst, zacc)
    return _tc_post(s2, cnt, r2, batch3, g)
